# baseline (speedup 1.0000x reference)
import functools

import jax
import jax.numpy as jnp
from jax import lax
from jax.experimental import pallas as pl
from jax.experimental.pallas import tpu as pltpu
from jax.experimental.pallas import tpu_sc as plsc

_SC_FRAC_NUM = 8
_TC_BLOCK = 2048
_CHUNK = 8
_NBUF = 8


@functools.cache
def _sc_route_copy(N, D, n_rows):
  info = plsc.get_sparse_core_info()
  NC, NS, L = info.num_cores, info.num_subcores, info.num_lanes
  NW = NC * NS
  assert n_rows % NW == 0 and D % L == 0
  rows_w = n_rows // NW
  CHUNK, NBUF = _CHUNK, _NBUF
  assert rows_w % CHUNK == 0 and rows_w % L == 0
  n_chunks = rows_w // CHUNK
  mesh = plsc.VectorSubcoreMesh(core_axis_name="c", subcore_axis_name="s")

  @functools.partial(
      pl.kernel,
      mesh=mesh,
      out_type=jax.ShapeDtypeStruct((N, D), jnp.float32),
      scratch_types=(
          [pltpu.VMEM((CHUNK, D), jnp.float32) for _ in range(NBUF)]
          + [pltpu.VMEM((1, D), jnp.float32),
             pltpu.VMEM((rows_w,), jnp.int32),
             pltpu.VMEM((L,), jnp.int32)]
          + [pltpu.SemaphoreType.DMA for _ in range(2 * NBUF)]
      ),
      compiler_params=pltpu.CompilerParams(needs_layout_passes=False),
  )
  def run(x_hbm, route_hbm, nr_hbm, out_hbm, *scr):
    bufs = scr[:NBUF]
    zeros_v, route_v, nr_v = scr[NBUF:NBUF + 3]
    si = scr[NBUF + 3:NBUF + 3 + NBUF]
    so = scr[NBUF + 3 + NBUF:]
    wid = lax.axis_index("s") * NC + lax.axis_index("c")
    base = wid * rows_w
    pltpu.sync_copy(route_hbm.at[pl.ds(base, rows_w)], route_v)
    pltpu.sync_copy(nr_hbm, nr_v)
    nr = nr_v[...]
    lane = lax.iota(jnp.int32, L)

    def start_in(g):
      src = x_hbm.at[pl.ds(base + g * CHUNK, CHUNK)]
      return pltpu.async_copy(src, bufs[g % NBUF], si[g % NBUF])

    in_h = {}
    out_h = {}
    for g in range(min(NBUF - 1, n_chunks)):
      in_h[g] = start_in(g)
    for g in range(n_chunks):
      b = g % NBUF
      in_h[g].wait()
      dst = out_hbm.at[pl.ds(base + g * CHUNK, CHUNK)]
      out_h[g] = pltpu.async_copy(bufs[b], dst, so[b])
      nxt = g + NBUF - 1
      if nxt < n_chunks:
        if nxt - NBUF >= 0:
          out_h[nxt - NBUF].wait()
        in_h[nxt] = start_in(nxt)
    for g in range(max(0, n_chunks - NBUF), n_chunks):
      out_h[g].wait()

    acc = jnp.zeros((L,), jnp.int32)
    for k in range(rows_w // L):
      v = route_v[pl.ds(k * L, L)]
      acc = acc + ((v < 0) | (v >= nr)).astype(jnp.int32)
    n_bad = jnp.sum(acc)

    @pl.when(n_bad > 0)
    def _fix():
      zf = jnp.zeros((L,), jnp.float32)
      for j in range(D // L):
        zeros_v[0, pl.ds(j * L, L)] = zf

      def per_group(k, c):
        v = route_v[pl.ds(k * L, L)]
        bad_f = ((v < 0) | (v >= nr)).astype(jnp.float32)

        @pl.when(jnp.sum(bad_f) > 0.0)
        def _fix_group():
          def per_row(i, c2):
            bad_i = jnp.sum(jnp.where(lane == i, bad_f, 0.0))

            @pl.when(bad_i > 0.0)
            def _zero_row():
              row = base + k * L + i
              pltpu.sync_copy(zeros_v, out_hbm.at[pl.ds(row, 1)])

            return c2

          lax.fori_loop(0, L, per_row, 0)

        return c

      lax.fori_loop(0, rows_w // L, per_group, 0)

  return run


@functools.cache
def _tc_route_copy(N, D, row0, bt):
  n_blocks = (N - row0) // bt
  assert row0 % bt == 0 and (N - row0) % bt == 0
  blk0 = row0 // bt

  def body(nr_ref, x_ref, r_ref, sc_ref, o_ref):
    del sc_ref
    nr = nr_ref[0]
    m = r_ref[...]
    keep = m.astype(jnp.uint32) < jnp.uint32(nr)
    o_ref[...] = jnp.where(keep, x_ref[...], 0.0)

  return pl.pallas_call(
      body,
      grid=(n_blocks,),
      in_specs=[
          pl.BlockSpec(memory_space=pltpu.SMEM),
          pl.BlockSpec((bt, D), lambda i: (blk0 + i, 0)),
          pl.BlockSpec((bt, 1), lambda i: (blk0 + i, 0)),
          pl.BlockSpec(memory_space=pl.MemorySpace.ANY),
      ],
      out_specs=pl.BlockSpec((bt, D), lambda i: (blk0 + i, 0)),
      out_shape=jax.ShapeDtypeStruct((N, D), jnp.float32),
      input_output_aliases={3: 0},
  )


def kernel(x, route, n_routes):
  B, T, D = x.shape
  N = B * T
  sc_rows = N * _SC_FRAC_NUM // 32
  x2 = x.reshape(N, D)
  rf = route.reshape(N).astype(jnp.int32)
  nr16 = jnp.full((16,), n_routes, dtype=jnp.int32)
  nr1 = jnp.full((1,), n_routes, dtype=jnp.int32)
  sc_out = _sc_route_copy(N, D, sc_rows)(x2, rf, nr16)
  out = _tc_route_copy(N, D, sc_rows, _TC_BLOCK)(
      nr1, x2, rf.reshape(N, 1), sc_out)
  return out.reshape(B, T, D)


# speedup vs baseline: 1.0046x; 1.0046x over previous
import functools

import jax
import jax.numpy as jnp
from jax import lax
from jax.experimental import pallas as pl
from jax.experimental.pallas import tpu as pltpu
from jax.experimental.pallas import tpu_sc as plsc

_SC_FRAC_NUM = 8
_TC_BLOCK = 2048
_CHUNK = 16
_NBUF = 4


@functools.cache
def _sc_route_copy(N, D, n_rows):
  info = plsc.get_sparse_core_info()
  NC, NS, L = info.num_cores, info.num_subcores, info.num_lanes
  NW = NC * NS
  assert n_rows % NW == 0 and D % L == 0
  rows_w = n_rows // NW
  CHUNK, NBUF = _CHUNK, _NBUF
  assert rows_w % CHUNK == 0 and rows_w % L == 0
  n_chunks = rows_w // CHUNK
  mesh = plsc.VectorSubcoreMesh(core_axis_name="c", subcore_axis_name="s")

  @functools.partial(
      pl.kernel,
      mesh=mesh,
      out_type=jax.ShapeDtypeStruct((N, D), jnp.float32),
      scratch_types=(
          [pltpu.VMEM((CHUNK, D), jnp.float32) for _ in range(NBUF)]
          + [pltpu.VMEM((1, D), jnp.float32),
             pltpu.VMEM((rows_w,), jnp.int32),
             pltpu.VMEM((L,), jnp.int32)]
          + [pltpu.SemaphoreType.DMA for _ in range(2 * NBUF)]
      ),
      compiler_params=pltpu.CompilerParams(needs_layout_passes=False),
  )
  def run(x_hbm, route_hbm, nr_hbm, out_hbm, *scr):
    bufs = scr[:NBUF]
    zeros_v, route_v, nr_v = scr[NBUF:NBUF + 3]
    si = scr[NBUF + 3:NBUF + 3 + NBUF]
    so = scr[NBUF + 3 + NBUF:]
    wid = lax.axis_index("s") * NC + lax.axis_index("c")
    base = wid * rows_w
    lane = lax.iota(jnp.int32, L)

    def start_in(g):
      src = x_hbm.at[pl.ds(base + g * CHUNK, CHUNK)]
      return pltpu.async_copy(src, bufs[g % NBUF], si[g % NBUF])

    in_h = {}
    out_h = {}
    for g in range(min(NBUF - 1, n_chunks)):
      in_h[g] = start_in(g)
    pltpu.sync_copy(route_hbm.at[pl.ds(base, rows_w)], route_v)
    pltpu.sync_copy(nr_hbm, nr_v)
    nr = nr_v[...]
    for g in range(n_chunks):
      b = g % NBUF
      in_h[g].wait()
      dst = out_hbm.at[pl.ds(base + g * CHUNK, CHUNK)]
      out_h[g] = pltpu.async_copy(bufs[b], dst, so[b])
      nxt = g + NBUF - 1
      if nxt < n_chunks:
        if nxt - NBUF >= 0:
          out_h[nxt - NBUF].wait()
        in_h[nxt] = start_in(nxt)
    for g in range(max(0, n_chunks - NBUF), n_chunks):
      out_h[g].wait()

    acc = jnp.zeros((L,), jnp.int32)
    for k in range(rows_w // L):
      v = route_v[pl.ds(k * L, L)]
      acc = acc + ((v < 0) | (v >= nr)).astype(jnp.int32)
    n_bad = jnp.sum(acc)

    @pl.when(n_bad > 0)
    def _fix():
      zf = jnp.zeros((L,), jnp.float32)
      for j in range(D // L):
        zeros_v[0, pl.ds(j * L, L)] = zf

      def per_group(k, c):
        v = route_v[pl.ds(k * L, L)]
        bad_f = ((v < 0) | (v >= nr)).astype(jnp.float32)

        @pl.when(jnp.sum(bad_f) > 0.0)
        def _fix_group():
          def per_row(i, c2):
            bad_i = jnp.sum(jnp.where(lane == i, bad_f, 0.0))

            @pl.when(bad_i > 0.0)
            def _zero_row():
              row = base + k * L + i
              pltpu.sync_copy(zeros_v, out_hbm.at[pl.ds(row, 1)])

            return c2

          lax.fori_loop(0, L, per_row, 0)

        return c

      lax.fori_loop(0, rows_w // L, per_group, 0)

  return run


@functools.cache
def _tc_route_copy(N, D, row0, bt):
  n_blocks = (N - row0) // bt
  assert row0 % bt == 0 and (N - row0) % bt == 0
  blk0 = row0 // bt

  def body(nr_ref, x_ref, r_ref, sc_ref, o_ref):
    del sc_ref
    nr = nr_ref[0]
    m = r_ref[...]
    keep = m.astype(jnp.uint32) < jnp.uint32(nr)
    o_ref[...] = jnp.where(keep, x_ref[...], 0.0)

  return pl.pallas_call(
      body,
      grid=(n_blocks,),
      in_specs=[
          pl.BlockSpec(memory_space=pltpu.SMEM),
          pl.BlockSpec((bt, D), lambda i: (blk0 + i, 0)),
          pl.BlockSpec((bt, 1), lambda i: (blk0 + i, 0)),
          pl.BlockSpec(memory_space=pl.MemorySpace.ANY),
      ],
      out_specs=pl.BlockSpec((bt, D), lambda i: (blk0 + i, 0)),
      out_shape=jax.ShapeDtypeStruct((N, D), jnp.float32),
      input_output_aliases={3: 0},
  )


def kernel(x, route, n_routes):
  B, T, D = x.shape
  N = B * T
  sc_rows = N * _SC_FRAC_NUM // 32
  x2 = x.reshape(N, D)
  rf = route.reshape(N).astype(jnp.int32)
  nr16 = jnp.full((16,), n_routes, dtype=jnp.int32)
  nr1 = jnp.full((1,), n_routes, dtype=jnp.int32)
  sc_out = _sc_route_copy(N, D, sc_rows)(x2, rf, nr16)
  out = _tc_route_copy(N, D, sc_rows, _TC_BLOCK)(
      nr1, x2, rf.reshape(N, 1), sc_out)
  return out.reshape(B, T, D)
